# baseline (speedup 1.0000x reference)
import functools

import jax
import jax.numpy as jnp
from jax import lax
from jax.experimental import pallas as pl
from jax.experimental.pallas import tpu as pltpu
from jax.experimental.pallas import tpu_sc as plsc

BATCH = 16384
DIM = 256
HALF = 128
LANES = 16
NC = 2
NS = 16
NW = NC * NS
PER_W = BATCH // NW
CHUNK = 64
NCHUNK = PER_W // CHUNK


def _score_one(t, hb, rb, tb, lane):
    acc = jnp.zeros((LANES,), jnp.float32)
    for k in range(HALF // LANES):
        lo = k * LANES
        rh = hb[t, pl.ds(lo, LANES)]
        ih = hb[t, pl.ds(HALF + lo, LANES)]
        rr = rb[t, pl.ds(lo, LANES)]
        ir = rb[t, pl.ds(HALF + lo, LANES)]
        rt = tb[t, pl.ds(lo, LANES)]
        it = tb[t, pl.ds(HALF + lo, LANES)]
        re_s = rr * rt + ir * it
        im_s = rr * it - ir * rt
        acc = acc + rh * re_s + ih * im_s
    for m in (8, 4, 2, 1):
        acc = acc + acc.at[lane ^ m].get(mode="promise_in_bounds")
    return acc


def _make_kernel():
    mesh = plsc.VectorSubcoreMesh(core_axis_name="c", subcore_axis_name="s")

    @functools.partial(
        pl.kernel,
        mesh=mesh,
        out_type=jax.ShapeDtypeStruct((NW, NCHUNK, CHUNK), jnp.float32),
        scratch_types=[
            pltpu.VMEM((3, NCHUNK, CHUNK), jnp.int32),
            pltpu.VMEM((2, CHUNK, DIM), jnp.float32),
            pltpu.VMEM((2, CHUNK, DIM), jnp.float32),
            pltpu.VMEM((2, CHUNK, DIM), jnp.float32),
            pltpu.VMEM((NCHUNK, CHUNK), jnp.float32),
            pltpu.SemaphoreType.DMA,
            pltpu.SemaphoreType.DMA,
        ],
    )
    def compl_ex_sc(idx_hbm, ent_hbm, rel_hbm, out_hbm,
                    idx_v, hbuf, rbuf, tbuf, out_v, sem0, sem1):
        wid = lax.axis_index("s") * NC + lax.axis_index("c")
        lane = lax.iota(jnp.int32, LANES)

        pltpu.sync_copy(idx_hbm.at[wid], idx_v)

        def fire(c, b, sem):
            pltpu.async_copy(ent_hbm.at[idx_v.at[0, c]], hbuf.at[b], sem)
            pltpu.async_copy(rel_hbm.at[idx_v.at[1, c]], rbuf.at[b], sem)
            pltpu.async_copy(ent_hbm.at[idx_v.at[2, c]], tbuf.at[b], sem)

        def drain(c, b, sem):
            pltpu.make_async_copy(
                ent_hbm.at[idx_v.at[0, c]], hbuf.at[b], sem).wait()
            pltpu.make_async_copy(
                rel_hbm.at[idx_v.at[1, c]], rbuf.at[b], sem).wait()
            pltpu.make_async_copy(
                ent_hbm.at[idx_v.at[2, c]], tbuf.at[b], sem).wait()

        fire(0, 0, sem0)

        def chunk_body(c, carry):
            nc = c + 1
            nxt_even = (nc < NCHUNK) & (nc % 2 == 0)
            nxt_odd = (nc < NCHUNK) & (nc % 2 == 1)
            pl.when(nxt_even)(lambda: fire(nc, 0, sem0))
            pl.when(nxt_odd)(lambda: fire(nc, 1, sem1))
            cur_even = c % 2 == 0
            pl.when(cur_even)(lambda: drain(c, 0, sem0))
            pl.when(~cur_even)(lambda: drain(c, 1, sem1))

            b = c % 2
            hb, rb, tb = hbuf.at[b], rbuf.at[b], tbuf.at[b]

            def group_body(g, carry2):
                res = jnp.zeros((LANES,), jnp.float32)
                for j in range(LANES):
                    s = _score_one(g * LANES + j, hb, rb, tb, lane)
                    res = jnp.where(lane == j, s, res)
                out_v[c, pl.ds(g * LANES, LANES)] = res
                return carry2

            lax.fori_loop(0, CHUNK // LANES, group_body, 0)
            return carry

        lax.fori_loop(0, NCHUNK, chunk_body, 0)
        pltpu.sync_copy(out_v, out_hbm.at[wid])

    return compl_ex_sc


_compl_ex = _make_kernel()


def kernel(triplet_idx, entity_embedding, relation_embedding):
    idx = triplet_idx.reshape(BATCH, 3).astype(jnp.int32)
    idx = idx.T.reshape(3, NW, NCHUNK, CHUNK).transpose(1, 0, 2, 3)
    out = _compl_ex(idx, entity_embedding, relation_embedding)
    return out.reshape(BATCH, 1)


# speedup vs baseline: 2.0420x; 2.0420x over previous
import functools

import jax
import jax.numpy as jnp
from jax import lax
from jax.experimental import pallas as pl
from jax.experimental.pallas import tpu as pltpu
from jax.experimental.pallas import tpu_sc as plsc

BATCH = 16384
DIM = 256
HALF = 128
LANES = 16
NC = 2
NS = 16
NW = NC * NS
PER_W = BATCH // NW
CHUNK = 64
NCHUNK = PER_W // CHUNK


def _score_one(t, hb, rb, tb, lane):
    acc = jnp.zeros((LANES,), jnp.float32)
    for k in range(HALF // LANES):
        lo = k * LANES
        rh = hb[t, pl.ds(lo, LANES)]
        ih = hb[t, pl.ds(HALF + lo, LANES)]
        rr = rb[t, pl.ds(lo, LANES)]
        ir = rb[t, pl.ds(HALF + lo, LANES)]
        rt = tb[t, pl.ds(lo, LANES)]
        it = tb[t, pl.ds(HALF + lo, LANES)]
        re_s = rr * rt + ir * it
        im_s = rr * it - ir * rt
        acc = acc + rh * re_s + ih * im_s
    for m in (8, 4, 2, 1):
        acc = acc + acc.at[lane ^ m].get(mode="promise_in_bounds")
    return acc


def _make_kernel():
    mesh = plsc.VectorSubcoreMesh(core_axis_name="c", subcore_axis_name="s")

    @functools.partial(
        pl.kernel,
        mesh=mesh,
        out_type=jax.ShapeDtypeStruct((NW, PER_W), jnp.float32),
        scratch_types=[
            pltpu.VMEM((3, NCHUNK, CHUNK), jnp.int32),
            pltpu.VMEM((CHUNK, DIM), jnp.float32),
            pltpu.VMEM((CHUNK, DIM), jnp.float32),
            pltpu.VMEM((CHUNK, DIM), jnp.float32),
            pltpu.VMEM((CHUNK, DIM), jnp.float32),
            pltpu.VMEM((CHUNK, DIM), jnp.float32),
            pltpu.VMEM((CHUNK, DIM), jnp.float32),
            pltpu.VMEM((PER_W + LANES,), jnp.float32),
            pltpu.SemaphoreType.DMA,
            pltpu.SemaphoreType.DMA,
        ],
    )
    def compl_ex_sc(idx_hbm, ent_hbm, rel_hbm, out_hbm,
                    idx_v, h0, r0, t0, h1, r1, t1, out_v, sem0, sem1):
        wid = lax.axis_index("s") * NC + lax.axis_index("c")
        lane = lax.iota(jnp.int32, LANES)
        hbufs = (h0, h1)
        rbufs = (r0, r1)
        tbufs = (t0, t1)
        sems = (sem0, sem1)

        pltpu.sync_copy(idx_hbm.at[wid], idx_v)

        def fire(c):
            s = sems[c % 2]
            return (
                pltpu.async_copy(ent_hbm.at[idx_v.at[0, c]], hbufs[c % 2], s),
                pltpu.async_copy(rel_hbm.at[idx_v.at[1, c]], rbufs[c % 2], s),
                pltpu.async_copy(ent_hbm.at[idx_v.at[2, c]], tbufs[c % 2], s),
            )

        inflight = fire(0)
        for c in range(NCHUNK):
            nxt = fire(c + 1) if c + 1 < NCHUNK else None
            for d in inflight:
                d.wait()
            inflight = nxt
            hb, rb, tb = hbufs[c % 2], rbufs[c % 2], tbufs[c % 2]

            def body(t, carry):
                s = _score_one(t, hb, rb, tb, lane)
                out_v[pl.ds(c * CHUNK + t, LANES)] = s
                return carry

            lax.fori_loop(0, CHUNK, body, 0, unroll=2)

        pltpu.sync_copy(out_v.at[pl.ds(0, PER_W)], out_hbm.at[wid])

    return compl_ex_sc


_compl_ex = _make_kernel()


def kernel(triplet_idx, entity_embedding, relation_embedding):
    idx = triplet_idx.reshape(BATCH, 3).astype(jnp.int32)
    idx = idx.T.reshape(3, NW, NCHUNK, CHUNK).transpose(1, 0, 2, 3)
    out = _compl_ex(idx, entity_embedding, relation_embedding)
    return out.reshape(BATCH, 1)
